# baseline (speedup 1.0000x reference)
import functools

import jax
import jax.numpy as jnp
from jax import lax
from jax.experimental import pallas as pl
from jax.experimental.pallas import tpu as pltpu
from jax.experimental.pallas import tpu_sc as plsc

_NUM_CODES = 1024
_DIM = 64
_B = 16
_T = 4096
_TB = 2048

_NC = 2
_NS = 16
_NW = _NC * _NS
_D_PER_W = _DIM // 2


def _tc_kernel(z_ref, emb_ref, idx_ref, loss_ref, perp_ref,
               counts_acc, loss_acc, enorm_acc):
    b = pl.program_id(0)
    j = pl.program_id(1)
    emb = emb_ref[...]

    @pl.when(jnp.logical_and(b == 0, j == 0))
    def _init():
        counts_acc[...] = jnp.zeros_like(counts_acc)
        loss_acc[...] = jnp.zeros_like(loss_acc)
        enorm_acc[...] = jnp.sum(emb * emb, axis=1, keepdims=True)

    z = z_ref[0]

    scores = jax.lax.dot_general(
        emb, z, (((1,), (0,)), ((), ())),
        preferred_element_type=jnp.float32,
        precision=jax.lax.Precision.DEFAULT)
    z_norm = jnp.sum(z * z, axis=0, keepdims=True)
    dist = (z_norm + enorm_acc[...]) - 2.0 * scores

    m = jnp.min(dist, axis=0, keepdims=True)
    one_hot = (dist == m).astype(jnp.float32)

    iota_row = jax.lax.broadcasted_iota(
        jnp.int32, (1, _NUM_CODES), 1).astype(jnp.float32)
    idx_f = jax.lax.dot_general(
        iota_row, one_hot, (((1,), (0,)), ((), ())),
        preferred_element_type=jnp.float32,
        precision=jax.lax.Precision.HIGHEST)
    idx_ref[0] = idx_f.astype(jnp.int32)

    loss_acc[...] += jnp.sum(m, axis=1, keepdims=True)
    counts_acc[...] += jnp.sum(one_hot, axis=1, keepdims=True)

    @pl.when(jnp.logical_and(b == pl.num_programs(0) - 1,
                             j == pl.num_programs(1) - 1))
    def _fin():
        total = jnp.float32(_B * _T * _DIM)
        loss_ref[...] = loss_acc[...] / total
        avg = counts_acc[...] / jnp.float32(_B * _T)
        ent = jnp.sum(avg * jnp.log(avg + 1e-10), axis=0, keepdims=True)
        perp_ref[...] = jnp.exp(-ent)


def _sc_gather(emb_t_hbm, idx_hbm, out_hbm, cols_v, idx_v, row_v, sem):
    c = lax.axis_index("c")
    s = lax.axis_index("s")
    wid = s * _NC + c
    b = wid // 2
    d0 = (wid % 2) * _D_PER_W
    pltpu.sync_copy(emb_t_hbm.at[pl.ds(d0 * _NUM_CODES, _D_PER_W * _NUM_CODES)],
                    cols_v)
    pltpu.sync_copy(idx_hbm.at[b, 0], idx_v)

    def d_body(dd, _):
        def t_body(t, _):
            iv = idx_v[pl.ds(t * 16, 16)] + dd * _NUM_CODES
            row_v[pl.ds(t * 16, 16)] = plsc.load_gather(cols_v, [iv])
            return 0
        lax.fori_loop(0, _T // 16, t_body, 0, unroll=8)
        pltpu.sync_copy(row_v, out_hbm.at[b, d0 + dd])
        return 0

    lax.fori_loop(0, _D_PER_W, d_body, 0)


def kernel(z_e, embedding):
    grid = (_B, _T // _TB)
    idx, loss, perp = pl.pallas_call(
        _tc_kernel,
        grid=grid,
        in_specs=[
            pl.BlockSpec((1, _DIM, _TB), lambda b, j: (b, 0, j)),
            pl.BlockSpec((_NUM_CODES, _DIM), lambda b, j: (0, 0)),
        ],
        out_specs=[
            pl.BlockSpec((1, 1, _TB), lambda b, j: (b, 0, j)),
            pl.BlockSpec((1, 1), lambda b, j: (0, 0)),
            pl.BlockSpec((1, 1), lambda b, j: (0, 0)),
        ],
        out_shape=[
            jax.ShapeDtypeStruct((_B, 1, _T), jnp.int32),
            jax.ShapeDtypeStruct((1, 1), jnp.float32),
            jax.ShapeDtypeStruct((1, 1), jnp.float32),
        ],
        scratch_shapes=[
            pltpu.VMEM((_NUM_CODES, 1), jnp.float32),
            pltpu.VMEM((1, 1), jnp.float32),
            pltpu.VMEM((_NUM_CODES, 1), jnp.float32),
        ],
        compiler_params=pltpu.CompilerParams(
            dimension_semantics=("arbitrary", "arbitrary")),
    )(z_e, embedding)

    emb_t = embedding.T.reshape(-1)

    mesh = plsc.VectorSubcoreMesh(
        core_axis_name="c", subcore_axis_name="s",
        num_cores=_NC, num_subcores=_NS)
    zq = pl.kernel(
        _sc_gather,
        mesh=mesh,
        compiler_params=pltpu.CompilerParams(needs_layout_passes=False),
        out_type=jax.ShapeDtypeStruct((_B, _DIM, _T), jnp.float32),
        scratch_types=[
            pltpu.VMEM((_D_PER_W * _NUM_CODES,), jnp.float32),
            pltpu.VMEM((_T,), jnp.int32),
            pltpu.VMEM((_T,), jnp.float32),
            pltpu.SemaphoreType.DMA,
        ],
    )(emb_t, idx)

    return zq, loss[0, 0], perp[0, 0]


# speedup vs baseline: 3.2004x; 3.2004x over previous
import jax
import jax.numpy as jnp
from jax.experimental import pallas as pl
from jax.experimental.pallas import tpu as pltpu

_NUM_CODES = 1024
_DIM = 64
_B = 16
_T = 4096
_TB = 2048


def _vq_kernel(z_ref, emb_ref, zq_ref, loss_ref, perp_ref,
               counts_acc, loss_acc, enorm_acc):
    b = pl.program_id(0)
    j = pl.program_id(1)
    emb = emb_ref[...]

    @pl.when(jnp.logical_and(b == 0, j == 0))
    def _init():
        counts_acc[...] = jnp.zeros_like(counts_acc)
        loss_acc[...] = jnp.zeros_like(loss_acc)
        enorm_acc[...] = jnp.sum(emb * emb, axis=1, keepdims=True)

    z = z_ref[0]

    scores = jax.lax.dot_general(
        emb, z, (((1,), (0,)), ((), ())),
        preferred_element_type=jnp.float32,
        precision=jax.lax.Precision.DEFAULT)
    z_norm = jnp.sum(z * z, axis=0, keepdims=True)
    dist = (z_norm + enorm_acc[...]) - 2.0 * scores

    m = jnp.min(dist, axis=0, keepdims=True)
    match = dist == m
    one_hot = match.astype(jnp.float32)

    zq = jax.lax.dot_general(
        emb, one_hot, (((0,), (0,)), ((), ())),
        preferred_element_type=jnp.float32,
        precision=jax.lax.Precision.DEFAULT)
    zq_ref[0] = zq

    loss_acc[...] += jnp.sum(m, axis=1, keepdims=True)
    counts_acc[...] += jnp.sum(one_hot, axis=1, keepdims=True)

    @pl.when(jnp.logical_and(b == pl.num_programs(0) - 1,
                             j == pl.num_programs(1) - 1))
    def _fin():
        total = jnp.float32(_B * _T * _DIM)
        loss_ref[...] = loss_acc[...] / total
        avg = counts_acc[...] / jnp.float32(_B * _T)
        ent = jnp.sum(avg * jnp.log(avg + 1e-10), axis=0, keepdims=True)
        perp_ref[...] = jnp.exp(-ent)


def kernel(z_e, embedding):
    grid = (_B, _T // _TB)
    zq, loss, perp = pl.pallas_call(
        _vq_kernel,
        grid=grid,
        in_specs=[
            pl.BlockSpec((1, _DIM, _TB), lambda b, j: (b, 0, j)),
            pl.BlockSpec((_NUM_CODES, _DIM), lambda b, j: (0, 0)),
        ],
        out_specs=[
            pl.BlockSpec((1, _DIM, _TB), lambda b, j: (b, 0, j)),
            pl.BlockSpec((1, 1), lambda b, j: (0, 0)),
            pl.BlockSpec((1, 1), lambda b, j: (0, 0)),
        ],
        out_shape=[
            jax.ShapeDtypeStruct((_B, _DIM, _T), jnp.float32),
            jax.ShapeDtypeStruct((1, 1), jnp.float32),
            jax.ShapeDtypeStruct((1, 1), jnp.float32),
        ],
        scratch_shapes=[
            pltpu.VMEM((_NUM_CODES, 1), jnp.float32),
            pltpu.VMEM((1, 1), jnp.float32),
            pltpu.VMEM((_NUM_CODES, 1), jnp.float32),
        ],
        compiler_params=pltpu.CompilerParams(
            dimension_semantics=("arbitrary", "arbitrary")),
    )(z_e, embedding)
    return zq, loss[0, 0], perp[0, 0]
